# Initial kernel scaffold; baseline (speedup 1.0000x reference)
import jax
import jax.numpy as jnp
from jax.experimental import pallas as pl


def kernel(x, edge_index, W1, b1, W2, b2, Wfc1, bfc1, Wfc2, bfc2):
    raise NotImplementedError("write your pallas kernel here")


# speedup vs baseline: 11.3014x; 11.3014x over previous
import functools

import jax
import jax.numpy as jnp
from jax import lax
from jax.experimental import pallas as pl
from jax.experimental.pallas import tpu as pltpu
from jax.experimental.pallas import tpu_sc as plsc

N = 10000
D = 128
HID2 = 256
EMB = 128
N_PAD = 10240
NC, NS = 2, 16
NW = NC * NS
K = 128
C_CH = 79
E_W = C_CH * K
E_PAD = NW * E_W
ROWS_T = N_PAD // NS
ZB = 16
BR = 512
GR = N_PAD // BR

def _deg_body(src_hbm, dst_hbm, out_hbm, sidx, didx, ones_v, stage_v, acc_s, acc_d):
    c = lax.axis_index("c")
    s = lax.axis_index("s")
    wid = c * NS + s
    for i in range(ROWS_T // 16):
        stage_v[pl.ds(i * 16, 16)] = jnp.zeros((16,), jnp.float32)
    for i in range(K // 16):
        ones_v[pl.ds(i * 16, 16)] = jnp.ones((16,), jnp.float32)
    base = s * ROWS_T
    pltpu.sync_copy(stage_v, acc_s.at[pl.ds(base, ROWS_T)])
    pltpu.sync_copy(stage_v, acc_d.at[pl.ds(base, ROWS_T)])
    plsc.subcore_barrier()
    pltpu.sync_copy(src_hbm.at[wid], sidx)
    pltpu.sync_copy(dst_hbm.at[wid], didx)

    def step(j, carry):
        pltpu.sync_copy(ones_v, acc_s.at[sidx.at[j]], add=True)
        pltpu.sync_copy(ones_v, acc_d.at[didx.at[j]], add=True)
        return carry

    lax.fori_loop(0, C_CH, step, 0)
    plsc.subcore_barrier()
    pltpu.sync_copy(acc_s.at[pl.ds(base, ROWS_T)], stage_v)
    pltpu.sync_copy(stage_v, out_hbm.at[c, 0, pl.ds(base, ROWS_T)])
    pltpu.sync_copy(acc_d.at[pl.ds(base, ROWS_T)], stage_v)
    pltpu.sync_copy(stage_v, out_hbm.at[c, 1, pl.ds(base, ROWS_T)])


def _agg_body(hs_hbm, src_hbm, dst_hbm, out_hbm, didx, sidx, rows_v, stage_v, acc, sem):
    c = lax.axis_index("c")
    s = lax.axis_index("s")
    wid = c * NS + s
    for r in range(ZB):
        for l in range(D // 16):
            stage_v[r, pl.ds(l * 16, 16)] = jnp.zeros((16,), jnp.float32)
    base = s * ROWS_T

    def zstep(b, carry):
        pltpu.sync_copy(stage_v, acc.at[pl.ds(base + b * ZB, ZB)])
        return carry

    lax.fori_loop(0, ROWS_T // ZB, zstep, 0)
    plsc.subcore_barrier()
    pltpu.sync_copy(dst_hbm.at[wid], didx)

    def step(j, carry):
        pltpu.sync_copy(src_hbm.at[wid, j], sidx)
        pltpu.async_copy(hs_hbm.at[sidx], rows_v, sem).wait()
        pltpu.sync_copy(rows_v, acc.at[didx.at[j]], add=True)
        return carry

    lax.fori_loop(0, C_CH, step, 0)
    plsc.subcore_barrier()

    def fstep(b, carry):
        pltpu.sync_copy(acc.at[pl.ds(base + b * ZB, ZB)], stage_v)
        pltpu.sync_copy(stage_v, out_hbm.at[c, pl.ds(base + b * ZB, ZB)])
        return carry

    lax.fori_loop(0, ROWS_T // ZB, fstep, 0)


@functools.cache
def _sc_kernels():
    mesh = plsc.VectorSubcoreMesh(core_axis_name="c", subcore_axis_name="s")
    deg = pl.kernel(
        _deg_body,
        out_type=jax.ShapeDtypeStruct((NC, 2, N_PAD), jnp.float32),
        mesh=mesh,
        scratch_types=[
            pltpu.VMEM((C_CH, K), jnp.int32),
            pltpu.VMEM((C_CH, K), jnp.int32),
            pltpu.VMEM((K,), jnp.float32),
            pltpu.VMEM((ROWS_T,), jnp.float32),
            pltpu.VMEM_SHARED((N_PAD,), jnp.float32),
            pltpu.VMEM_SHARED((N_PAD,), jnp.float32),
        ],
    )
    agg = pl.kernel(
        _agg_body,
        out_type=jax.ShapeDtypeStruct((NC, N_PAD, D), jnp.float32),
        mesh=mesh,
        scratch_types=[
            pltpu.VMEM((C_CH, K), jnp.int32),
            pltpu.VMEM((K,), jnp.int32),
            pltpu.VMEM((K, D), jnp.float32),
            pltpu.VMEM((ZB, D), jnp.float32),
            pltpu.VMEM_SHARED((N_PAD, D), jnp.float32),
            pltpu.SemaphoreType.DMA,
        ],
    )
    return deg, agg


def _mm1_body(deg_ref, x_ref, w_ref, o_ref):
    nsrc = lax.rsqrt(jnp.maximum(deg_ref[0, 0] + deg_ref[1, 0], 1.0))
    o_ref[...] = (
        jnp.dot(x_ref[...], w_ref[...], preferred_element_type=jnp.float32) * nsrc
    )


def _mid_body(deg_ref, p_ref, b1_ref, w2_ref, o_ref):
    nsrc = lax.rsqrt(jnp.maximum(deg_ref[0, 0] + deg_ref[1, 0], 1.0))
    ndst = lax.rsqrt(jnp.maximum(deg_ref[0, 1] + deg_ref[1, 1], 1.0))
    agg = p_ref[0] + p_ref[1]
    h1 = jnp.maximum(agg * ndst + b1_ref[...], 0.0)
    o_ref[...] = (
        jnp.dot(h1, w2_ref[...], preferred_element_type=jnp.float32) * nsrc
    )


def _tail_body(deg_ref, p_ref, b2_ref, wf1_ref, bf1_ref, wf2_ref, bf2_ref,
               o_ref, mx_ref):
    r = pl.program_id(0)
    ndst = lax.rsqrt(jnp.maximum(deg_ref[0, 1] + deg_ref[1, 1], 1.0))
    h2 = jnp.maximum((p_ref[0] + p_ref[1]) * ndst + b2_ref[...], 0.0)
    rows = r * BR + lax.broadcasted_iota(jnp.int32, (BR, D), 0)
    h2 = jnp.where(rows < N, h2, -jnp.inf)
    mx = jnp.max(h2, axis=0, keepdims=True)

    @pl.when(r == 0)
    def _():
        mx_ref[...] = mx

    @pl.when(r > 0)
    def _():
        mx_ref[...] = jnp.maximum(mx_ref[...], mx)

    @pl.when(r == GR - 1)
    def _():
        pooled = mx_ref[...]
        h = jnp.maximum(
            jnp.dot(pooled, wf1_ref[...], preferred_element_type=jnp.float32)
            + bf1_ref[...], 0.0)
        o_ref[...] = (
            jnp.dot(h, wf2_ref[...], preferred_element_type=jnp.float32)
            + bf2_ref[...])


_DEG_SPEC = pl.BlockSpec((NC, 2, BR, 1), lambda r: (0, 0, r, 0))
_W_SPEC = pl.BlockSpec((D, D), lambda r: (0, 0))
_ROW_SPEC = pl.BlockSpec((BR, D), lambda r: (r, 0))
_PARTS_SPEC = pl.BlockSpec((NC, BR, D), lambda r: (0, r, 0))
_B_SPEC = pl.BlockSpec((1, D), lambda r: (0, 0))

_mm1_call = pl.pallas_call(
    _mm1_body,
    grid=(GR,),
    in_specs=[_DEG_SPEC, _ROW_SPEC, _W_SPEC],
    out_specs=_ROW_SPEC,
    out_shape=jax.ShapeDtypeStruct((N_PAD, D), jnp.float32),
)

_mid_call = pl.pallas_call(
    _mid_body,
    grid=(GR,),
    in_specs=[_DEG_SPEC, _PARTS_SPEC, _B_SPEC, _W_SPEC],
    out_specs=_ROW_SPEC,
    out_shape=jax.ShapeDtypeStruct((N_PAD, D), jnp.float32),
)

_tail_call = pl.pallas_call(
    _tail_body,
    grid=(GR,),
    in_specs=[
        _DEG_SPEC,
        _PARTS_SPEC,
        _B_SPEC,
        pl.BlockSpec((D, HID2), lambda r: (0, 0)),
        pl.BlockSpec((1, HID2), lambda r: (0, 0)),
        pl.BlockSpec((HID2, EMB), lambda r: (0, 0)),
        pl.BlockSpec((1, EMB), lambda r: (0, 0)),
    ],
    out_specs=pl.BlockSpec((1, EMB), lambda r: (0, 0)),
    out_shape=jax.ShapeDtypeStruct((1, EMB), jnp.float32),
    scratch_shapes=[pltpu.VMEM((1, D), jnp.float32)],
)


def kernel(x, edge_index, W1, b1, W2, b2, Wfc1, bfc1, Wfc2, bfc2):
    src = edge_index[0].astype(jnp.int32)
    dst = edge_index[1].astype(jnp.int32)
    n_extra = E_PAD - src.shape[0]
    pad_idx = N + (jnp.arange(n_extra, dtype=jnp.int32) % (N_PAD - N))
    src_p = jnp.concatenate([src, pad_idx]).reshape(NW, C_CH, K)
    dst_p = jnp.concatenate([dst, pad_idx]).reshape(NW, C_CH, K)
    x_p = jnp.pad(x, ((0, N_PAD - N), (0, 0)))

    deg_k, agg_k = _sc_kernels()
    deg4 = deg_k(src_p, dst_p).reshape(NC, 2, N_PAD, 1)
    hs1 = _mm1_call(deg4, x_p, W1)
    parts1 = agg_k(hs1, src_p, dst_p)
    hs2 = _mid_call(deg4, parts1, b1.reshape(1, D), W2)
    parts2 = agg_k(hs2, src_p, dst_p)
    return _tail_call(deg4, parts2, b2.reshape(1, D), Wfc1,
                      bfc1.reshape(1, HID2), Wfc2, bfc2.reshape(1, EMB))


# speedup vs baseline: 15.0330x; 1.3302x over previous
import functools

import jax
import jax.numpy as jnp
from jax import lax
from jax.experimental import pallas as pl
from jax.experimental.pallas import tpu as pltpu
from jax.experimental.pallas import tpu_sc as plsc

N = 10000
D = 128
HID2 = 256
EMB = 128
N_PAD = 10240
NC, NS = 2, 16
NW = NC * NS
K = 128
C_CH = 80
C_H = C_CH // 2
NB = 2
G_SUP = C_H // NB
E_W = C_CH * K
E_PAD = NW * E_W
ROWS_T = N_PAD // NS
ZB = 16
FK = 64
BR = 512
GR = N_PAD // BR


def _deg_body(src_hbm, dst_hbm, out_hbm, sidx, didx, ones_v, stage_v, acc_s, acc_d):
    c = lax.axis_index("c")
    s = lax.axis_index("s")
    wid = c * NS + s
    for i in range(ROWS_T // 16):
        stage_v[pl.ds(i * 16, 16)] = jnp.zeros((16,), jnp.float32)
    for i in range(K // 16):
        ones_v[pl.ds(i * 16, 16)] = jnp.ones((16,), jnp.float32)
    base = s * ROWS_T
    pltpu.sync_copy(stage_v, acc_s.at[pl.ds(base, ROWS_T)])
    pltpu.sync_copy(stage_v, acc_d.at[pl.ds(base, ROWS_T)])
    plsc.subcore_barrier()
    pltpu.sync_copy(src_hbm.at[wid], sidx)
    pltpu.sync_copy(dst_hbm.at[wid], didx)

    def step(j, carry):
        pltpu.sync_copy(ones_v, acc_s.at[sidx.at[j]], add=True)
        pltpu.sync_copy(ones_v, acc_d.at[didx.at[j]], add=True)
        return carry

    lax.fori_loop(0, C_CH, step, 0)
    plsc.subcore_barrier()
    pltpu.sync_copy(acc_s.at[pl.ds(base, ROWS_T)], stage_v)
    pltpu.sync_copy(stage_v, out_hbm.at[c, 0, pl.ds(base, ROWS_T)])
    pltpu.sync_copy(acc_d.at[pl.ds(base, ROWS_T)], stage_v)
    pltpu.sync_copy(stage_v, out_hbm.at[c, 1, pl.ds(base, ROWS_T)])


def _agg_body(hs_hbm, src_hbm, dst_hbm, zeros_hbm, out_hbm, sidx, didx, rows_v, acc,
              g0, g1, s0, s1, zsem, fsem):
    gsem = [g0, g1]
    ssem = [s0, s1]
    c = lax.axis_index("c")
    s = lax.axis_index("s")
    wid = c * NS + s
    base = s * ROWS_T
    pltpu.async_copy(zeros_hbm, acc.at[pl.ds(base, ROWS_T)], zsem)

    def half_pass(h):
        off = h * C_H
        pltpu.sync_copy(src_hbm.at[wid, pl.ds(off, C_H)], sidx)
        pltpu.sync_copy(dst_hbm.at[wid, pl.ds(off, C_H)], didx)
        for b in range(NB):
            pltpu.async_copy(hs_hbm.at[sidx.at[b]], rows_v.at[b], gsem[b])
        if h == 0:
            pltpu.make_async_copy(zeros_hbm, acc.at[pl.ds(base, ROWS_T)], zsem).wait()
            plsc.subcore_barrier()

        def super_step(g, carry):
            for b in range(NB):
                j = g * NB + b
                pltpu.make_async_copy(
                    hs_hbm.at[sidx.at[j]], rows_v.at[b], gsem[b]).wait()
                pltpu.async_copy(rows_v.at[b], acc.at[didx.at[j]], ssem[b], add=True)
            for b in range(NB):
                j = g * NB + b
                pltpu.make_async_copy(
                    rows_v.at[b], acc.at[didx.at[j]], ssem[b]).wait()

                @pl.when(g < G_SUP - 1)
                def _():
                    pltpu.async_copy(
                        hs_hbm.at[sidx.at[j + NB]], rows_v.at[b], gsem[b])
            return carry

        lax.fori_loop(0, G_SUP, super_step, 0)

    half_pass(0)
    half_pass(1)
    plsc.subcore_barrier()
    for i in range(ROWS_T // FK):
        pltpu.async_copy(acc.at[pl.ds(base + i * FK, FK)],
                         out_hbm.at[c, pl.ds(base + i * FK, FK)], fsem)
    for i in range(ROWS_T // FK):
        pltpu.make_async_copy(acc.at[pl.ds(base + i * FK, FK)],
                              out_hbm.at[c, pl.ds(base + i * FK, FK)], fsem).wait()


@functools.cache
def _sc_kernels():
    mesh = plsc.VectorSubcoreMesh(core_axis_name="c", subcore_axis_name="s")
    deg = pl.kernel(
        _deg_body,
        out_type=jax.ShapeDtypeStruct((NC, 2, N_PAD), jnp.float32),
        mesh=mesh,
        scratch_types=[
            pltpu.VMEM((C_CH, K), jnp.int32),
            pltpu.VMEM((C_CH, K), jnp.int32),
            pltpu.VMEM((K,), jnp.float32),
            pltpu.VMEM((ROWS_T,), jnp.float32),
            pltpu.VMEM_SHARED((N_PAD,), jnp.float32),
            pltpu.VMEM_SHARED((N_PAD,), jnp.float32),
        ],
    )
    agg = pl.kernel(
        _agg_body,
        out_type=jax.ShapeDtypeStruct((NC, N_PAD, D), jnp.float32),
        mesh=mesh,
        scratch_types=[
            pltpu.VMEM((C_H, K), jnp.int32),
            pltpu.VMEM((C_H, K), jnp.int32),
            pltpu.VMEM((NB, K, D), jnp.float32),
            pltpu.VMEM_SHARED((N_PAD, D), jnp.float32),
        ] + [pltpu.SemaphoreType.DMA] * 6,
    )
    return deg, agg


def _mm1_body(deg_ref, x_ref, w_ref, o_ref):
    nsrc = lax.rsqrt(jnp.maximum(deg_ref[0, 0] + deg_ref[1, 0], 1.0))
    o_ref[...] = (
        jnp.dot(x_ref[...], w_ref[...], preferred_element_type=jnp.float32) * nsrc
    )


def _mid_body(deg_ref, p_ref, b1_ref, w2_ref, o_ref):
    nsrc = lax.rsqrt(jnp.maximum(deg_ref[0, 0] + deg_ref[1, 0], 1.0))
    ndst = lax.rsqrt(jnp.maximum(deg_ref[0, 1] + deg_ref[1, 1], 1.0))
    agg = p_ref[0] + p_ref[1]
    h1 = jnp.maximum(agg * ndst + b1_ref[...], 0.0)
    o_ref[...] = (
        jnp.dot(h1, w2_ref[...], preferred_element_type=jnp.float32) * nsrc
    )


def _tail_body(deg_ref, p_ref, b2_ref, wf1_ref, bf1_ref, wf2_ref, bf2_ref,
               o_ref, mx_ref):
    r = pl.program_id(0)
    ndst = lax.rsqrt(jnp.maximum(deg_ref[0, 1] + deg_ref[1, 1], 1.0))
    h2 = jnp.maximum((p_ref[0] + p_ref[1]) * ndst + b2_ref[...], 0.0)
    rows = r * BR + lax.broadcasted_iota(jnp.int32, (BR, D), 0)
    h2 = jnp.where(rows < N, h2, -jnp.inf)
    mx = jnp.max(h2, axis=0, keepdims=True)

    @pl.when(r == 0)
    def _():
        mx_ref[...] = mx

    @pl.when(r > 0)
    def _():
        mx_ref[...] = jnp.maximum(mx_ref[...], mx)

    @pl.when(r == GR - 1)
    def _():
        pooled = mx_ref[...]
        h = jnp.maximum(
            jnp.dot(pooled, wf1_ref[...], preferred_element_type=jnp.float32)
            + bf1_ref[...], 0.0)
        o_ref[...] = (
            jnp.dot(h, wf2_ref[...], preferred_element_type=jnp.float32)
            + bf2_ref[...])


_DEG_SPEC = pl.BlockSpec((NC, 2, BR, 1), lambda r: (0, 0, r, 0))
_W_SPEC = pl.BlockSpec((D, D), lambda r: (0, 0))
_ROW_SPEC = pl.BlockSpec((BR, D), lambda r: (r, 0))
_PARTS_SPEC = pl.BlockSpec((NC, BR, D), lambda r: (0, r, 0))
_B_SPEC = pl.BlockSpec((1, D), lambda r: (0, 0))

_mm1_call = pl.pallas_call(
    _mm1_body,
    grid=(GR,),
    in_specs=[_DEG_SPEC, _ROW_SPEC, _W_SPEC],
    out_specs=_ROW_SPEC,
    out_shape=jax.ShapeDtypeStruct((N_PAD, D), jnp.float32),
)

_mid_call = pl.pallas_call(
    _mid_body,
    grid=(GR,),
    in_specs=[_DEG_SPEC, _PARTS_SPEC, _B_SPEC, _W_SPEC],
    out_specs=_ROW_SPEC,
    out_shape=jax.ShapeDtypeStruct((N_PAD, D), jnp.float32),
)

_tail_call = pl.pallas_call(
    _tail_body,
    grid=(GR,),
    in_specs=[
        _DEG_SPEC,
        _PARTS_SPEC,
        _B_SPEC,
        pl.BlockSpec((D, HID2), lambda r: (0, 0)),
        pl.BlockSpec((1, HID2), lambda r: (0, 0)),
        pl.BlockSpec((HID2, EMB), lambda r: (0, 0)),
        pl.BlockSpec((1, EMB), lambda r: (0, 0)),
    ],
    out_specs=pl.BlockSpec((1, EMB), lambda r: (0, 0)),
    out_shape=jax.ShapeDtypeStruct((1, EMB), jnp.float32),
    scratch_shapes=[pltpu.VMEM((1, D), jnp.float32)],
)


def kernel(x, edge_index, W1, b1, W2, b2, Wfc1, bfc1, Wfc2, bfc2):
    src = edge_index[0].astype(jnp.int32)
    dst = edge_index[1].astype(jnp.int32)
    n_extra = E_PAD - src.shape[0]
    pad_idx = N + (jnp.arange(n_extra, dtype=jnp.int32) % (N_PAD - N))
    src_p = jnp.concatenate([src, pad_idx]).reshape(NW, C_CH, K)
    dst_p = jnp.concatenate([dst, pad_idx]).reshape(NW, C_CH, K)
    x_p = jnp.pad(x, ((0, N_PAD - N), (0, 0)))

    deg_k, agg_k = _sc_kernels()
    zrows = jnp.zeros((ROWS_T, D), jnp.float32)
    deg4 = deg_k(src_p, dst_p).reshape(NC, 2, N_PAD, 1)
    hs1 = _mm1_call(deg4, x_p, W1)
    parts1 = agg_k(hs1, src_p, dst_p, zrows)
    hs2 = _mid_call(deg4, parts1, b1.reshape(1, D), W2)
    parts2 = agg_k(hs2, src_p, dst_p, zrows)
    return _tail_call(deg4, parts2, b2.reshape(1, D), Wfc1,
                      bfc1.reshape(1, HID2), Wfc2, bfc2.reshape(1, EMB))


# speedup vs baseline: 18.1363x; 1.2064x over previous
import functools

import jax
import jax.numpy as jnp
from jax import lax
from jax.experimental import pallas as pl
from jax.experimental.pallas import tpu as pltpu
from jax.experimental.pallas import tpu_sc as plsc

N = 10000
D = 128
HID2 = 256
EMB = 128
N_PAD = 10240
NC, NS = 2, 16
NW = NC * NS
K = 128
KG = 64
C_CH = 80
C_H = C_CH // 2
NB = 4
T_H = C_H * K // KG
G_SUP = T_H // NB
E_W = C_CH * K
E_PAD = NW * E_W
ROWS_T = N_PAD // NS
ZB = 16
FK = 64
BR = 512
GR = N_PAD // BR


def _deg_body(src_hbm, dst_hbm, out_hbm, sidx, didx, ones_v, stage_v, acc_s, acc_d, dsem_s, dsem_d):
    c = lax.axis_index("c")
    s = lax.axis_index("s")
    wid = c * NS + s
    for i in range(ROWS_T // 16):
        stage_v[pl.ds(i * 16, 16)] = jnp.zeros((16,), jnp.float32)
    for i in range(K // 16):
        ones_v[pl.ds(i * 16, 16)] = jnp.ones((16,), jnp.float32)
    base = s * ROWS_T
    pltpu.sync_copy(stage_v, acc_s.at[pl.ds(base, ROWS_T)])
    pltpu.sync_copy(stage_v, acc_d.at[pl.ds(base, ROWS_T)])
    plsc.subcore_barrier()
    pltpu.sync_copy(src_hbm.at[wid], sidx)
    pltpu.sync_copy(dst_hbm.at[wid], didx)

    def step(j, carry):
        @pl.when(j >= 8)
        def _():
            pltpu.make_async_copy(ones_v, acc_s.at[sidx.at[j - 8]], dsem_s).wait()
            pltpu.make_async_copy(ones_v, acc_d.at[didx.at[j - 8]], dsem_d).wait()

        pltpu.async_copy(ones_v, acc_s.at[sidx.at[j]], dsem_s, add=True)
        pltpu.async_copy(ones_v, acc_d.at[didx.at[j]], dsem_d, add=True)
        return carry

    lax.fori_loop(0, C_CH, step, 0)

    def dstep(j, carry):
        pltpu.make_async_copy(ones_v, acc_s.at[sidx.at[j]], dsem_s).wait()
        pltpu.make_async_copy(ones_v, acc_d.at[didx.at[j]], dsem_d).wait()
        return carry

    lax.fori_loop(C_CH - 8, C_CH, dstep, 0)
    plsc.subcore_barrier()
    pltpu.sync_copy(acc_s.at[pl.ds(base, ROWS_T)], stage_v)
    pltpu.sync_copy(stage_v, out_hbm.at[c, 0, pl.ds(base, ROWS_T)])
    pltpu.sync_copy(acc_d.at[pl.ds(base, ROWS_T)], stage_v)
    pltpu.sync_copy(stage_v, out_hbm.at[c, 1, pl.ds(base, ROWS_T)])


def _agg_body(hs_hbm, src_hbm, dst_hbm, zeros_hbm, out_hbm, sidx, didx, rows_v, acc,
              g0, g1, g2, g3, s0, s1, s2, s3, zsem, fsem):
    gsem = [g0, g1, g2, g3]
    ssem = [s0, s1, s2, s3]
    c = lax.axis_index("c")
    s = lax.axis_index("s")
    wid = c * NS + s
    base = s * ROWS_T
    pltpu.async_copy(zeros_hbm, acc.at[pl.ds(base, ROWS_T)], zsem)

    def gidx(g, b):
        return sidx.at[g * (NB // 2) + b // 2, pl.ds((b % 2) * KG, KG)]

    def half_pass(h):
        pltpu.sync_copy(src_hbm.at[wid, pl.ds(h * C_H, C_H)], sidx)
        pltpu.sync_copy(dst_hbm.at[wid, pl.ds(h * T_H, T_H)], didx)
        for b in range(NB):
            pltpu.async_copy(hs_hbm.at[gidx(0, b)], rows_v.at[b], gsem[b])
        if h == 0:
            pltpu.make_async_copy(zeros_hbm, acc.at[pl.ds(base, ROWS_T)], zsem).wait()
            plsc.subcore_barrier()

        def super_step(g, carry):
            for b in range(NB):
                t = g * NB + b
                pltpu.make_async_copy(
                    hs_hbm.at[gidx(g, b)], rows_v.at[b], gsem[b]).wait()
                pltpu.async_copy(rows_v.at[b], acc.at[didx.at[t]], ssem[b], add=True)
            for b in range(NB):
                t = g * NB + b
                pltpu.make_async_copy(
                    rows_v.at[b], acc.at[didx.at[t]], ssem[b]).wait()

                @pl.when(g < G_SUP - 1)
                def _():
                    pltpu.async_copy(
                        hs_hbm.at[gidx(g + 1, b)], rows_v.at[b], gsem[b])
            return carry

        lax.fori_loop(0, G_SUP, super_step, 0)

    half_pass(0)
    half_pass(1)
    plsc.subcore_barrier()
    for i in range(ROWS_T // FK):
        pltpu.async_copy(acc.at[pl.ds(base + i * FK, FK)],
                         out_hbm.at[c, pl.ds(base + i * FK, FK)], fsem)
    for i in range(ROWS_T // FK):
        pltpu.make_async_copy(acc.at[pl.ds(base + i * FK, FK)],
                              out_hbm.at[c, pl.ds(base + i * FK, FK)], fsem).wait()


@functools.cache
def _sc_kernels():
    mesh = plsc.VectorSubcoreMesh(core_axis_name="c", subcore_axis_name="s")
    deg = pl.kernel(
        _deg_body,
        out_type=jax.ShapeDtypeStruct((NC, 2, N_PAD), jnp.float32),
        mesh=mesh,
        scratch_types=[
            pltpu.VMEM((C_CH, K), jnp.int32),
            pltpu.VMEM((C_CH, K), jnp.int32),
            pltpu.VMEM((K,), jnp.float32),
            pltpu.VMEM((ROWS_T,), jnp.float32),
            pltpu.VMEM_SHARED((N_PAD,), jnp.float32),
            pltpu.VMEM_SHARED((N_PAD,), jnp.float32),
            pltpu.SemaphoreType.DMA,
            pltpu.SemaphoreType.DMA,
        ],
    )
    agg = pl.kernel(
        _agg_body,
        out_type=jax.ShapeDtypeStruct((NC, N_PAD, D), jnp.float32),
        mesh=mesh,
        scratch_types=[
            pltpu.VMEM((C_H, K), jnp.int32),
            pltpu.VMEM((T_H, KG), jnp.int32),
            pltpu.VMEM((NB, KG, D), jnp.float32),
            pltpu.VMEM_SHARED((N_PAD, D), jnp.float32),
        ] + [pltpu.SemaphoreType.DMA] * 10,
    )
    return deg, agg


def _mm1_body(deg_ref, x_ref, w_ref, o_ref):
    nsrc = lax.rsqrt(jnp.maximum(deg_ref[0, 0] + deg_ref[1, 0], 1.0))
    o_ref[...] = (
        jnp.dot(x_ref[...], w_ref[...], preferred_element_type=jnp.float32) * nsrc
    )


def _mid_body(deg_ref, p_ref, b1_ref, w2_ref, o_ref):
    nsrc = lax.rsqrt(jnp.maximum(deg_ref[0, 0] + deg_ref[1, 0], 1.0))
    ndst = lax.rsqrt(jnp.maximum(deg_ref[0, 1] + deg_ref[1, 1], 1.0))
    agg = p_ref[0] + p_ref[1]
    h1 = jnp.maximum(agg * ndst + b1_ref[...], 0.0)
    o_ref[...] = (
        jnp.dot(h1, w2_ref[...], preferred_element_type=jnp.float32) * nsrc
    )


def _tail_body(deg_ref, p_ref, b2_ref, wf1_ref, bf1_ref, wf2_ref, bf2_ref,
               o_ref, mx_ref):
    r = pl.program_id(0)
    ndst = lax.rsqrt(jnp.maximum(deg_ref[0, 1] + deg_ref[1, 1], 1.0))
    h2 = jnp.maximum((p_ref[0] + p_ref[1]) * ndst + b2_ref[...], 0.0)
    rows = r * BR + lax.broadcasted_iota(jnp.int32, (BR, D), 0)
    h2 = jnp.where(rows < N, h2, -jnp.inf)
    mx = jnp.max(h2, axis=0, keepdims=True)

    @pl.when(r == 0)
    def _():
        mx_ref[...] = mx

    @pl.when(r > 0)
    def _():
        mx_ref[...] = jnp.maximum(mx_ref[...], mx)

    @pl.when(r == GR - 1)
    def _():
        pooled = mx_ref[...]
        h = jnp.maximum(
            jnp.dot(pooled, wf1_ref[...], preferred_element_type=jnp.float32)
            + bf1_ref[...], 0.0)
        o_ref[...] = (
            jnp.dot(h, wf2_ref[...], preferred_element_type=jnp.float32)
            + bf2_ref[...])


_DEG_SPEC = pl.BlockSpec((NC, 2, BR, 1), lambda r: (0, 0, r, 0))
_W_SPEC = pl.BlockSpec((D, D), lambda r: (0, 0))
_ROW_SPEC = pl.BlockSpec((BR, D), lambda r: (r, 0))
_PARTS_SPEC = pl.BlockSpec((NC, BR, D), lambda r: (0, r, 0))
_B_SPEC = pl.BlockSpec((1, D), lambda r: (0, 0))

_mm1_call = pl.pallas_call(
    _mm1_body,
    grid=(GR,),
    in_specs=[_DEG_SPEC, _ROW_SPEC, _W_SPEC],
    out_specs=_ROW_SPEC,
    out_shape=jax.ShapeDtypeStruct((N_PAD, D), jnp.float32),
)

_mid_call = pl.pallas_call(
    _mid_body,
    grid=(GR,),
    in_specs=[_DEG_SPEC, _PARTS_SPEC, _B_SPEC, _W_SPEC],
    out_specs=_ROW_SPEC,
    out_shape=jax.ShapeDtypeStruct((N_PAD, D), jnp.float32),
)

_tail_call = pl.pallas_call(
    _tail_body,
    grid=(GR,),
    in_specs=[
        _DEG_SPEC,
        _PARTS_SPEC,
        _B_SPEC,
        pl.BlockSpec((D, HID2), lambda r: (0, 0)),
        pl.BlockSpec((1, HID2), lambda r: (0, 0)),
        pl.BlockSpec((HID2, EMB), lambda r: (0, 0)),
        pl.BlockSpec((1, EMB), lambda r: (0, 0)),
    ],
    out_specs=pl.BlockSpec((1, EMB), lambda r: (0, 0)),
    out_shape=jax.ShapeDtypeStruct((1, EMB), jnp.float32),
    scratch_shapes=[pltpu.VMEM((1, D), jnp.float32)],
)


def kernel(x, edge_index, W1, b1, W2, b2, Wfc1, bfc1, Wfc2, bfc2):
    src = edge_index[0].astype(jnp.int32)
    dst = edge_index[1].astype(jnp.int32)
    n_extra = E_PAD - src.shape[0]
    pad_idx = N + (jnp.arange(n_extra, dtype=jnp.int32) % (N_PAD - N))
    src_p = jnp.concatenate([src, pad_idx]).reshape(NW, C_CH, K)
    dst_p = jnp.concatenate([dst, pad_idx]).reshape(NW, C_CH, K)
    x_p = jnp.pad(x, ((0, N_PAD - N), (0, 0)))

    deg_k, agg_k = _sc_kernels()
    zrows = jnp.zeros((ROWS_T, D), jnp.float32)
    dst_a = dst_p.reshape(NW, 2 * T_H, KG)
    deg4 = deg_k(src_p, dst_p).reshape(NC, 2, N_PAD, 1)
    hs1 = _mm1_call(deg4, x_p, W1)
    parts1 = agg_k(hs1, src_p, dst_a, zrows)
    hs2 = _mid_call(deg4, parts1, b1.reshape(1, D), W2)
    parts2 = agg_k(hs2, src_p, dst_a, zrows)
    return _tail_call(deg4, parts2, b2.reshape(1, D), Wfc1,
                      bfc1.reshape(1, HID2), Wfc2, bfc2.reshape(1, EMB))


# speedup vs baseline: 18.3439x; 1.0114x over previous
import functools

import jax
import jax.numpy as jnp
from jax import lax
from jax.experimental import pallas as pl
from jax.experimental.pallas import tpu as pltpu
from jax.experimental.pallas import tpu_sc as plsc

N = 10000
D = 128
HID2 = 256
EMB = 128
N_PAD = 10240
NC, NS = 2, 16
NW = NC * NS
K = 128
KG = 64
C_CH = 80
NQ = 4
C_Q = C_CH // NQ
NB = 4
T_Q = C_Q * K // KG
G_SUP = T_Q // NB
E_W = C_CH * K
E_PAD = NW * E_W
ROWS_T = N_PAD // NS
ZB = 16
FK = 64
BR = 512
GR = N_PAD // BR


def _deg_body(src_hbm, dst_hbm, out_hbm, sidx, didx, ones_v, stage_v, acc_s, acc_d, dsem_s, dsem_d):
    c = lax.axis_index("c")
    s = lax.axis_index("s")
    wid = c * NS + s
    for i in range(ROWS_T // 16):
        stage_v[pl.ds(i * 16, 16)] = jnp.zeros((16,), jnp.float32)
    for i in range(K // 16):
        ones_v[pl.ds(i * 16, 16)] = jnp.ones((16,), jnp.float32)
    base = s * ROWS_T
    pltpu.sync_copy(stage_v, acc_s.at[pl.ds(base, ROWS_T)])
    pltpu.sync_copy(stage_v, acc_d.at[pl.ds(base, ROWS_T)])
    plsc.subcore_barrier()
    pltpu.sync_copy(src_hbm.at[wid], sidx)
    pltpu.sync_copy(dst_hbm.at[wid], didx)

    def step(j, carry):
        @pl.when(j >= 8)
        def _():
            pltpu.make_async_copy(ones_v, acc_s.at[sidx.at[j - 8]], dsem_s).wait()
            pltpu.make_async_copy(ones_v, acc_d.at[didx.at[j - 8]], dsem_d).wait()

        pltpu.async_copy(ones_v, acc_s.at[sidx.at[j]], dsem_s, add=True)
        pltpu.async_copy(ones_v, acc_d.at[didx.at[j]], dsem_d, add=True)
        return carry

    lax.fori_loop(0, C_CH, step, 0)

    def dstep(j, carry):
        pltpu.make_async_copy(ones_v, acc_s.at[sidx.at[j]], dsem_s).wait()
        pltpu.make_async_copy(ones_v, acc_d.at[didx.at[j]], dsem_d).wait()
        return carry

    lax.fori_loop(C_CH - 8, C_CH, dstep, 0)
    plsc.subcore_barrier()
    pltpu.sync_copy(acc_s.at[pl.ds(base, ROWS_T)], stage_v)
    pltpu.sync_copy(stage_v, out_hbm.at[c, 0, pl.ds(base, ROWS_T)])
    pltpu.sync_copy(acc_d.at[pl.ds(base, ROWS_T)], stage_v)
    pltpu.sync_copy(stage_v, out_hbm.at[c, 1, pl.ds(base, ROWS_T)])


def _agg_body(hs_hbm, src_hbm, dst_hbm, zeros_hbm, out_hbm, sidx_a, sidx_b,
              didx_a, didx_b, rows_v, acc,
              g0, g1, g2, g3, s0, s1, s2, s3, zsem, fsem, is0, is1, id0, id1):
    gsem = [g0, g1, g2, g3]
    ssem = [s0, s1, s2, s3]
    isem_s = [is0, is1]
    isem_d = [id0, id1]
    sidx2 = [sidx_a, sidx_b]
    didx2 = [didx_a, didx_b]
    c = lax.axis_index("c")
    s = lax.axis_index("s")
    wid = c * NS + s
    base = s * ROWS_T
    pltpu.async_copy(zeros_hbm, acc.at[pl.ds(base, ROWS_T)], zsem)

    def idx_load(q):
        p = q % 2
        pltpu.async_copy(src_hbm.at[wid, q], sidx2[p], isem_s[p])
        pltpu.async_copy(dst_hbm.at[wid, q], didx2[p], isem_d[p])

    def idx_wait(q):
        p = q % 2
        pltpu.make_async_copy(src_hbm.at[wid, q], sidx2[p], isem_s[p]).wait()
        pltpu.make_async_copy(dst_hbm.at[wid, q], didx2[p], isem_d[p]).wait()

    def gidx(p, g, b):
        return sidx2[p].at[g * (NB // 2) + b // 2, pl.ds((b % 2) * KG, KG)]

    idx_load(0)
    idx_wait(0)
    for b in range(NB):
        pltpu.async_copy(hs_hbm.at[gidx(0, 0, b)], rows_v.at[b], gsem[b])
    idx_load(1)
    pltpu.make_async_copy(zeros_hbm, acc.at[pl.ds(base, ROWS_T)], zsem).wait()
    plsc.subcore_barrier()

    def quarter_pass(q):
        p = q % 2

        def super_step(g, carry):
            for b in range(NB):
                t = g * NB + b
                pltpu.make_async_copy(
                    hs_hbm.at[gidx(p, g, b)], rows_v.at[b], gsem[b]).wait()
                pltpu.async_copy(rows_v.at[b], acc.at[didx2[p].at[t]],
                                 ssem[b], add=True)
            for b in range(NB):
                t = g * NB + b
                pltpu.make_async_copy(
                    rows_v.at[b], acc.at[didx2[p].at[t]], ssem[b]).wait()

                @pl.when(g < G_SUP - 1)
                def _():
                    pltpu.async_copy(
                        hs_hbm.at[gidx(p, g + 1, b)], rows_v.at[b], gsem[b])
            return carry

        lax.fori_loop(0, G_SUP - 1, super_step, 0)
        g = G_SUP - 1
        pn = 1 - p
        if q + 1 < NQ:
            idx_wait(q + 1)
        for b in range(NB):
            t = g * NB + b
            pltpu.make_async_copy(
                hs_hbm.at[gidx(p, g, b)], rows_v.at[b], gsem[b]).wait()
            pltpu.async_copy(rows_v.at[b], acc.at[didx2[p].at[t]],
                             ssem[b], add=True)
        for b in range(NB):
            t = g * NB + b
            pltpu.make_async_copy(
                rows_v.at[b], acc.at[didx2[p].at[t]], ssem[b]).wait()
            if q + 1 < NQ:
                pltpu.async_copy(
                    hs_hbm.at[gidx(pn, 0, b)], rows_v.at[b], gsem[b])
        if q + 2 < NQ:
            idx_load(q + 2)

    for q in range(NQ):
        quarter_pass(q)
    plsc.subcore_barrier()
    for i in range(ROWS_T // FK):
        pltpu.async_copy(acc.at[pl.ds(base + i * FK, FK)],
                         out_hbm.at[c, pl.ds(base + i * FK, FK)], fsem)
    for i in range(ROWS_T // FK):
        pltpu.make_async_copy(acc.at[pl.ds(base + i * FK, FK)],
                              out_hbm.at[c, pl.ds(base + i * FK, FK)], fsem).wait()


@functools.cache
def _sc_kernels():
    mesh = plsc.VectorSubcoreMesh(core_axis_name="c", subcore_axis_name="s")
    deg = pl.kernel(
        _deg_body,
        out_type=jax.ShapeDtypeStruct((NC, 2, N_PAD), jnp.float32),
        mesh=mesh,
        scratch_types=[
            pltpu.VMEM((C_CH, K), jnp.int32),
            pltpu.VMEM((C_CH, K), jnp.int32),
            pltpu.VMEM((K,), jnp.float32),
            pltpu.VMEM((ROWS_T,), jnp.float32),
            pltpu.VMEM_SHARED((N_PAD,), jnp.float32),
            pltpu.VMEM_SHARED((N_PAD,), jnp.float32),
            pltpu.SemaphoreType.DMA,
            pltpu.SemaphoreType.DMA,
        ],
    )
    agg = pl.kernel(
        _agg_body,
        out_type=jax.ShapeDtypeStruct((NC, N_PAD, D), jnp.float32),
        mesh=mesh,
        scratch_types=[
            pltpu.VMEM((C_Q, K), jnp.int32),
            pltpu.VMEM((C_Q, K), jnp.int32),
            pltpu.VMEM((T_Q, KG), jnp.int32),
            pltpu.VMEM((T_Q, KG), jnp.int32),
            pltpu.VMEM((NB, KG, D), jnp.float32),
            pltpu.VMEM_SHARED((N_PAD, D), jnp.float32),
        ] + [pltpu.SemaphoreType.DMA] * 14,
    )
    return deg, agg


def _mm1_body(deg_ref, x_ref, w_ref, o_ref):
    nsrc = lax.rsqrt(jnp.maximum(deg_ref[0, 0] + deg_ref[1, 0], 1.0))
    o_ref[...] = (
        jnp.dot(x_ref[...], w_ref[...], preferred_element_type=jnp.float32) * nsrc
    )


def _mid_body(deg_ref, p_ref, b1_ref, w2_ref, o_ref):
    nsrc = lax.rsqrt(jnp.maximum(deg_ref[0, 0] + deg_ref[1, 0], 1.0))
    ndst = lax.rsqrt(jnp.maximum(deg_ref[0, 1] + deg_ref[1, 1], 1.0))
    agg = p_ref[0] + p_ref[1]
    h1 = jnp.maximum(agg * ndst + b1_ref[...], 0.0)
    o_ref[...] = (
        jnp.dot(h1, w2_ref[...], preferred_element_type=jnp.float32) * nsrc
    )


def _tail_body(deg_ref, p_ref, b2_ref, wf1_ref, bf1_ref, wf2_ref, bf2_ref,
               o_ref, mx_ref):
    r = pl.program_id(0)
    ndst = lax.rsqrt(jnp.maximum(deg_ref[0, 1] + deg_ref[1, 1], 1.0))
    h2 = jnp.maximum((p_ref[0] + p_ref[1]) * ndst + b2_ref[...], 0.0)
    rows = r * BR + lax.broadcasted_iota(jnp.int32, (BR, D), 0)
    h2 = jnp.where(rows < N, h2, -jnp.inf)
    mx = jnp.max(h2, axis=0, keepdims=True)

    @pl.when(r == 0)
    def _():
        mx_ref[...] = mx

    @pl.when(r > 0)
    def _():
        mx_ref[...] = jnp.maximum(mx_ref[...], mx)

    @pl.when(r == GR - 1)
    def _():
        pooled = mx_ref[...]
        h = jnp.maximum(
            jnp.dot(pooled, wf1_ref[...], preferred_element_type=jnp.float32)
            + bf1_ref[...], 0.0)
        o_ref[...] = (
            jnp.dot(h, wf2_ref[...], preferred_element_type=jnp.float32)
            + bf2_ref[...])


_DEG_SPEC = pl.BlockSpec((NC, 2, BR, 1), lambda r: (0, 0, r, 0))
_W_SPEC = pl.BlockSpec((D, D), lambda r: (0, 0))
_ROW_SPEC = pl.BlockSpec((BR, D), lambda r: (r, 0))
_PARTS_SPEC = pl.BlockSpec((NC, BR, D), lambda r: (0, r, 0))
_B_SPEC = pl.BlockSpec((1, D), lambda r: (0, 0))

_mm1_call = pl.pallas_call(
    _mm1_body,
    grid=(GR,),
    in_specs=[_DEG_SPEC, _ROW_SPEC, _W_SPEC],
    out_specs=_ROW_SPEC,
    out_shape=jax.ShapeDtypeStruct((N_PAD, D), jnp.float32),
)

_mid_call = pl.pallas_call(
    _mid_body,
    grid=(GR,),
    in_specs=[_DEG_SPEC, _PARTS_SPEC, _B_SPEC, _W_SPEC],
    out_specs=_ROW_SPEC,
    out_shape=jax.ShapeDtypeStruct((N_PAD, D), jnp.float32),
)

_tail_call = pl.pallas_call(
    _tail_body,
    grid=(GR,),
    in_specs=[
        _DEG_SPEC,
        _PARTS_SPEC,
        _B_SPEC,
        pl.BlockSpec((D, HID2), lambda r: (0, 0)),
        pl.BlockSpec((1, HID2), lambda r: (0, 0)),
        pl.BlockSpec((HID2, EMB), lambda r: (0, 0)),
        pl.BlockSpec((1, EMB), lambda r: (0, 0)),
    ],
    out_specs=pl.BlockSpec((1, EMB), lambda r: (0, 0)),
    out_shape=jax.ShapeDtypeStruct((1, EMB), jnp.float32),
    scratch_shapes=[pltpu.VMEM((1, D), jnp.float32)],
)


def kernel(x, edge_index, W1, b1, W2, b2, Wfc1, bfc1, Wfc2, bfc2):
    src = edge_index[0].astype(jnp.int32)
    dst = edge_index[1].astype(jnp.int32)
    n_extra = E_PAD - src.shape[0]
    pad_idx = N + (jnp.arange(n_extra, dtype=jnp.int32) % (N_PAD - N))
    src_p = jnp.concatenate([src, pad_idx]).reshape(NW, C_CH, K)
    dst_p = jnp.concatenate([dst, pad_idx]).reshape(NW, C_CH, K)
    x_p = jnp.pad(x, ((0, N_PAD - N), (0, 0)))

    deg_k, agg_k = _sc_kernels()
    zrows = jnp.zeros((ROWS_T, D), jnp.float32)
    src_a = src_p.reshape(NW, NQ, C_Q, K)
    dst_a = dst_p.reshape(NW, NQ, T_Q, KG)
    deg4 = deg_k(src_p, dst_p).reshape(NC, 2, N_PAD, 1)
    hs1 = _mm1_call(deg4, x_p, W1)
    parts1 = agg_k(hs1, src_a, dst_a, zrows)
    hs2 = _mid_call(deg4, parts1, b1.reshape(1, D), W2)
    parts2 = agg_k(hs2, src_a, dst_a, zrows)
    return _tail_call(deg4, parts2, b2.reshape(1, D), Wfc1,
                      bfc1.reshape(1, HID2), Wfc2, bfc2.reshape(1, EMB))


# speedup vs baseline: 19.3595x; 1.0554x over previous
import functools

import jax
import jax.numpy as jnp
from jax import lax
from jax.experimental import pallas as pl
from jax.experimental.pallas import tpu as pltpu
from jax.experimental.pallas import tpu_sc as plsc

N = 10000
D = 128
HID2 = 256
EMB = 128
N_PAD = 10240
NC, NS = 2, 16
NW = NC * NS
K = 128
KG = 64
C_CH = 80
NQ = 5
C_Q = C_CH // NQ
NB = 4
T_Q = C_Q * K // KG
G_SUP = T_Q // NB
E_W = C_CH * K
E_PAD = NW * E_W
ROWS_T = N_PAD // NS
FK = 64
BR = 1000
GR = N // BR


def _deg_body(src_hbm, dst_hbm, out_hbm, sidx, didx, ones_v, stage_v, acc_s, acc_d, dsem_s, dsem_d):
    c = lax.axis_index("c")
    s = lax.axis_index("s")
    wid = c * NS + s
    for i in range(ROWS_T // 16):
        stage_v[pl.ds(i * 16, 16)] = jnp.zeros((16,), jnp.float32)
    for i in range(K // 16):
        ones_v[pl.ds(i * 16, 16)] = jnp.ones((16,), jnp.float32)
    base = s * ROWS_T
    pltpu.sync_copy(stage_v, acc_s.at[pl.ds(base, ROWS_T)])
    pltpu.sync_copy(stage_v, acc_d.at[pl.ds(base, ROWS_T)])
    plsc.subcore_barrier()
    pltpu.sync_copy(src_hbm.at[wid], sidx)
    pltpu.sync_copy(dst_hbm.at[wid], didx)

    def step(j, carry):
        @pl.when(j >= 8)
        def _():
            pltpu.make_async_copy(ones_v, acc_s.at[sidx.at[j - 8]], dsem_s).wait()
            pltpu.make_async_copy(ones_v, acc_d.at[didx.at[j - 8]], dsem_d).wait()

        pltpu.async_copy(ones_v, acc_s.at[sidx.at[j]], dsem_s, add=True)
        pltpu.async_copy(ones_v, acc_d.at[didx.at[j]], dsem_d, add=True)
        return carry

    lax.fori_loop(0, C_CH, step, 0)

    def dstep(j, carry):
        pltpu.make_async_copy(ones_v, acc_s.at[sidx.at[j]], dsem_s).wait()
        pltpu.make_async_copy(ones_v, acc_d.at[didx.at[j]], dsem_d).wait()
        return carry

    lax.fori_loop(C_CH - 8, C_CH, dstep, 0)
    plsc.subcore_barrier()
    pltpu.sync_copy(acc_s.at[pl.ds(base, ROWS_T)], stage_v)
    pltpu.sync_copy(stage_v, out_hbm.at[c, 0, pl.ds(base, ROWS_T)])
    pltpu.sync_copy(acc_d.at[pl.ds(base, ROWS_T)], stage_v)
    pltpu.sync_copy(stage_v, out_hbm.at[c, 1, pl.ds(base, ROWS_T)])


def _agg_body(hs_hbm, src_hbm, dst_hbm, zeros_hbm, out_hbm, sidx_a, sidx_b,
              didx_a, didx_b, rows_v, acc,
              g0, g1, g2, g3, s0, s1, s2, s3, zsem, fsem, is0, is1, id0, id1):
    gsem = [g0, g1, g2, g3]
    ssem = [s0, s1, s2, s3]
    isem_s = [is0, is1]
    isem_d = [id0, id1]
    sidx2 = [sidx_a, sidx_b]
    didx2 = [didx_a, didx_b]
    c = lax.axis_index("c")
    s = lax.axis_index("s")
    wid = c * NS + s
    base = s * ROWS_T
    pltpu.async_copy(zeros_hbm, acc.at[pl.ds(base, ROWS_T)], zsem)

    def idx_load(q):
        p = q % 2
        pltpu.async_copy(src_hbm.at[wid, pl.ds(q * C_Q, C_Q)], sidx2[p], isem_s[p])
        pltpu.async_copy(dst_hbm.at[wid, q], didx2[p], isem_d[p])

    def idx_wait(q):
        p = q % 2
        pltpu.make_async_copy(
            src_hbm.at[wid, pl.ds(q * C_Q, C_Q)], sidx2[p], isem_s[p]).wait()
        pltpu.make_async_copy(dst_hbm.at[wid, q], didx2[p], isem_d[p]).wait()

    def gidx(p, g, b):
        return sidx2[p].at[g * (NB // 2) + b // 2, pl.ds((b % 2) * KG, KG)]

    idx_load(0)
    idx_wait(0)
    for b in range(NB):
        pltpu.async_copy(hs_hbm.at[gidx(0, 0, b)], rows_v.at[b], gsem[b])
    idx_load(1)
    pltpu.make_async_copy(zeros_hbm, acc.at[pl.ds(base, ROWS_T)], zsem).wait()
    plsc.subcore_barrier()

    def quarter_pass(q):
        p = q % 2

        def super_step(g, carry):
            for b in range(NB):
                t = g * NB + b
                pltpu.make_async_copy(
                    hs_hbm.at[gidx(p, g, b)], rows_v.at[b], gsem[b]).wait()
                pltpu.async_copy(rows_v.at[b], acc.at[didx2[p].at[t]],
                                 ssem[b], add=True)
            for b in range(NB):
                t = g * NB + b
                pltpu.make_async_copy(
                    rows_v.at[b], acc.at[didx2[p].at[t]], ssem[b]).wait()

                @pl.when(g < G_SUP - 1)
                def _():
                    pltpu.async_copy(
                        hs_hbm.at[gidx(p, g + 1, b)], rows_v.at[b], gsem[b])
            return carry

        lax.fori_loop(0, G_SUP - 1, super_step, 0)
        g = G_SUP - 1
        pn = 1 - p
        if q + 1 < NQ:
            idx_wait(q + 1)
        for b in range(NB):
            t = g * NB + b
            pltpu.make_async_copy(
                hs_hbm.at[gidx(p, g, b)], rows_v.at[b], gsem[b]).wait()
            pltpu.async_copy(rows_v.at[b], acc.at[didx2[p].at[t]],
                             ssem[b], add=True)
        for b in range(NB):
            t = g * NB + b
            pltpu.make_async_copy(
                rows_v.at[b], acc.at[didx2[p].at[t]], ssem[b]).wait()
            if q + 1 < NQ:
                pltpu.async_copy(
                    hs_hbm.at[gidx(pn, 0, b)], rows_v.at[b], gsem[b])
        if q + 2 < NQ:
            idx_load(q + 2)

    for q in range(NQ):
        quarter_pass(q)
    plsc.subcore_barrier()
    for i in range(ROWS_T // FK):
        pltpu.async_copy(acc.at[pl.ds(base + i * FK, FK)],
                         out_hbm.at[c, pl.ds(base + i * FK, FK)], fsem)
    for i in range(ROWS_T // FK):
        pltpu.make_async_copy(acc.at[pl.ds(base + i * FK, FK)],
                              out_hbm.at[c, pl.ds(base + i * FK, FK)], fsem).wait()


@functools.cache
def _sc_kernels():
    mesh = plsc.VectorSubcoreMesh(core_axis_name="c", subcore_axis_name="s")
    deg = pl.kernel(
        _deg_body,
        out_type=jax.ShapeDtypeStruct((NC, 2, N_PAD), jnp.float32),
        mesh=mesh,
        scratch_types=[
            pltpu.VMEM((C_CH, K), jnp.int32),
            pltpu.VMEM((C_CH, K), jnp.int32),
            pltpu.VMEM((K,), jnp.float32),
            pltpu.VMEM((ROWS_T,), jnp.float32),
            pltpu.VMEM_SHARED((N_PAD,), jnp.float32),
            pltpu.VMEM_SHARED((N_PAD,), jnp.float32),
            pltpu.SemaphoreType.DMA,
            pltpu.SemaphoreType.DMA,
        ],
    )
    agg = pl.kernel(
        _agg_body,
        out_type=jax.ShapeDtypeStruct((NC, N_PAD, D), jnp.float32),
        mesh=mesh,
        scratch_types=[
            pltpu.VMEM((C_Q, K), jnp.int32),
            pltpu.VMEM((C_Q, K), jnp.int32),
            pltpu.VMEM((T_Q, KG), jnp.int32),
            pltpu.VMEM((T_Q, KG), jnp.int32),
            pltpu.VMEM((NB, KG, D), jnp.float32),
            pltpu.VMEM_SHARED((N_PAD, D), jnp.float32),
        ] + [pltpu.SemaphoreType.DMA] * 14,
    )
    return deg, agg


def _mm1_body(deg_ref, x_ref, w_ref, o_ref):
    nsrc = lax.rsqrt(jnp.maximum(deg_ref[0, 0] + deg_ref[1, 0], 1.0))
    o_ref[...] = (
        jnp.dot(x_ref[...], w_ref[...], preferred_element_type=jnp.float32) * nsrc
    )


def _mid_body(deg_ref, p_ref, b1_ref, w2_ref, o_ref):
    nsrc = lax.rsqrt(jnp.maximum(deg_ref[0, 0] + deg_ref[1, 0], 1.0))
    ndst = lax.rsqrt(jnp.maximum(deg_ref[0, 1] + deg_ref[1, 1], 1.0))
    agg = p_ref[0] + p_ref[1]
    h1 = jnp.maximum(agg * ndst + b1_ref[...], 0.0)
    o_ref[...] = (
        jnp.dot(h1, w2_ref[...], preferred_element_type=jnp.float32) * nsrc
    )


def _tail_body(deg_ref, p_ref, b2_ref, wf1_ref, bf1_ref, wf2_ref, bf2_ref,
               o_ref, mx_ref):
    r = pl.program_id(0)
    ndst = lax.rsqrt(jnp.maximum(deg_ref[0, 1] + deg_ref[1, 1], 1.0))
    h2 = jnp.maximum((p_ref[0] + p_ref[1]) * ndst + b2_ref[...], 0.0)
    mx = jnp.max(h2, axis=0, keepdims=True)

    @pl.when(r == 0)
    def _():
        mx_ref[...] = mx

    @pl.when(r > 0)
    def _():
        mx_ref[...] = jnp.maximum(mx_ref[...], mx)

    @pl.when(r == GR - 1)
    def _():
        pooled = mx_ref[...]
        h = jnp.maximum(
            jnp.dot(pooled, wf1_ref[...], preferred_element_type=jnp.float32)
            + bf1_ref[...], 0.0)
        o_ref[...] = (
            jnp.dot(h, wf2_ref[...], preferred_element_type=jnp.float32)
            + bf2_ref[...])


_DEG_SPEC = pl.BlockSpec((NC, 2, BR, 1), lambda r: (0, 0, r, 0))
_W_SPEC = pl.BlockSpec((D, D), lambda r: (0, 0))
_ROW_SPEC = pl.BlockSpec((BR, D), lambda r: (r, 0))
_PARTS_SPEC = pl.BlockSpec((NC, BR, D), lambda r: (0, r, 0))
_B_SPEC = pl.BlockSpec((1, D), lambda r: (0, 0))

_mm1_call = pl.pallas_call(
    _mm1_body,
    grid=(GR,),
    in_specs=[_DEG_SPEC, _ROW_SPEC, _W_SPEC],
    out_specs=_ROW_SPEC,
    out_shape=jax.ShapeDtypeStruct((N, D), jnp.float32),
)

_mid_call = pl.pallas_call(
    _mid_body,
    grid=(GR,),
    in_specs=[_DEG_SPEC, _PARTS_SPEC, _B_SPEC, _W_SPEC],
    out_specs=_ROW_SPEC,
    out_shape=jax.ShapeDtypeStruct((N, D), jnp.float32),
)

_tail_call = pl.pallas_call(
    _tail_body,
    grid=(GR,),
    in_specs=[
        _DEG_SPEC,
        _PARTS_SPEC,
        _B_SPEC,
        pl.BlockSpec((D, HID2), lambda r: (0, 0)),
        pl.BlockSpec((1, HID2), lambda r: (0, 0)),
        pl.BlockSpec((HID2, EMB), lambda r: (0, 0)),
        pl.BlockSpec((1, EMB), lambda r: (0, 0)),
    ],
    out_specs=pl.BlockSpec((1, EMB), lambda r: (0, 0)),
    out_shape=jax.ShapeDtypeStruct((1, EMB), jnp.float32),
    scratch_shapes=[pltpu.VMEM((1, D), jnp.float32)],
)


def kernel(x, edge_index, W1, b1, W2, b2, Wfc1, bfc1, Wfc2, bfc2):
    src = edge_index[0].astype(jnp.int32)
    dst = edge_index[1].astype(jnp.int32)
    n_extra = E_PAD - src.shape[0]
    spread = jnp.arange(n_extra, dtype=jnp.int32) % 240
    pad_hi = N + spread
    src_deg = jnp.concatenate([src, pad_hi]).reshape(NW, C_CH, K)
    dst_p = jnp.concatenate([dst, pad_hi]).reshape(NW, C_CH, K)
    src_a = jnp.concatenate([src, spread]).reshape(NW, C_CH, K)
    dst_a = dst_p.reshape(NW, NQ, T_Q, KG)

    deg_k, agg_k = _sc_kernels()
    zrows = jnp.zeros((ROWS_T, D), jnp.float32)
    deg4 = deg_k(src_deg, dst_p).reshape(NC, 2, N_PAD, 1)
    hs1 = _mm1_call(deg4, x, W1)
    parts1 = agg_k(hs1, src_a, dst_a, zrows)
    hs2 = _mid_call(deg4, parts1, b1.reshape(1, D), W2)
    parts2 = agg_k(hs2, src_a, dst_a, zrows)
    return _tail_call(deg4, parts2, b2.reshape(1, D), Wfc1,
                      bfc1.reshape(1, HID2), Wfc2, bfc2.reshape(1, EMB))


# speedup vs baseline: 19.7714x; 1.0213x over previous
import functools

import numpy as np
import jax
import jax.numpy as jnp
from jax import lax
from jax.experimental import pallas as pl
from jax.experimental.pallas import tpu as pltpu
from jax.experimental.pallas import tpu_sc as plsc

N = 10000
D = 128
HID2 = 256
EMB = 128
N_PAD = 10240
NC, NS = 2, 16
NW = NC * NS
K = 128
KG = 64
C_CH = 80
NQ = 5
C_Q = C_CH // NQ
NB = 4
T_Q = C_Q * K // KG
G_SUP = T_Q // NB
E_W = C_CH * K
E_PAD = NW * E_W
ROWS_T = N_PAD // NS
FK = 64
BR = 2000
GR = N // BR


def _deg_body(src_hbm, dst_hbm, out_hbm, sidx, didx, ones_v, stage_v, acc_s, acc_d, dsem_s, dsem_d):
    c = lax.axis_index("c")
    s = lax.axis_index("s")
    wid = c * NS + s
    for i in range(ROWS_T // 16):
        stage_v[pl.ds(i * 16, 16)] = jnp.zeros((16,), jnp.float32)
    for i in range(K // 16):
        ones_v[pl.ds(i * 16, 16)] = jnp.ones((16,), jnp.float32)
    base = s * ROWS_T
    pltpu.sync_copy(stage_v, acc_s.at[pl.ds(base, ROWS_T)])
    pltpu.sync_copy(stage_v, acc_d.at[pl.ds(base, ROWS_T)])
    plsc.subcore_barrier()
    pltpu.sync_copy(src_hbm.at[wid], sidx)
    pltpu.sync_copy(dst_hbm.at[wid], didx)

    def step(j, carry):
        @pl.when(j >= 8)
        def _():
            pltpu.make_async_copy(ones_v, acc_s.at[sidx.at[j - 8]], dsem_s).wait()
            pltpu.make_async_copy(ones_v, acc_d.at[didx.at[j - 8]], dsem_d).wait()

        pltpu.async_copy(ones_v, acc_s.at[sidx.at[j]], dsem_s, add=True)
        pltpu.async_copy(ones_v, acc_d.at[didx.at[j]], dsem_d, add=True)
        return carry

    lax.fori_loop(0, C_CH, step, 0)

    def dstep(j, carry):
        pltpu.make_async_copy(ones_v, acc_s.at[sidx.at[j]], dsem_s).wait()
        pltpu.make_async_copy(ones_v, acc_d.at[didx.at[j]], dsem_d).wait()
        return carry

    lax.fori_loop(C_CH - 8, C_CH, dstep, 0)
    plsc.subcore_barrier()
    pltpu.sync_copy(acc_s.at[pl.ds(base, ROWS_T)], stage_v)
    pltpu.sync_copy(stage_v, out_hbm.at[c, 0, pl.ds(base, ROWS_T)])
    pltpu.sync_copy(acc_d.at[pl.ds(base, ROWS_T)], stage_v)
    pltpu.sync_copy(stage_v, out_hbm.at[c, 1, pl.ds(base, ROWS_T)])


def _agg_body(hs_hbm, src_hbm, dst_hbm, zeros_hbm, out_hbm, sidx_a, sidx_b,
              didx_a, didx_b, rows_v, acc,
              g0, g1, g2, g3, s0, s1, s2, s3, zsem, fsem, is0, is1, id0, id1):
    gsem = [g0, g1, g2, g3]
    ssem = [s0, s1, s2, s3]
    isem_s = [is0, is1]
    isem_d = [id0, id1]
    sidx2 = [sidx_a, sidx_b]
    didx2 = [didx_a, didx_b]
    c = lax.axis_index("c")
    s = lax.axis_index("s")
    wid = c * NS + s
    base = s * ROWS_T
    pltpu.async_copy(zeros_hbm, acc.at[pl.ds(base, ROWS_T)], zsem)

    def idx_load(q):
        p = q % 2
        pltpu.async_copy(src_hbm.at[wid, pl.ds(q * C_Q, C_Q)], sidx2[p], isem_s[p])
        pltpu.async_copy(dst_hbm.at[wid, pl.ds(q * C_Q, C_Q)], didx2[p], isem_d[p])

    def idx_wait(q):
        p = q % 2
        pltpu.make_async_copy(
            src_hbm.at[wid, pl.ds(q * C_Q, C_Q)], sidx2[p], isem_s[p]).wait()
        pltpu.make_async_copy(
            dst_hbm.at[wid, pl.ds(q * C_Q, C_Q)], didx2[p], isem_d[p]).wait()

    def gidx(p, g, b):
        return sidx2[p].at[g * (NB // 2) + b // 2, pl.ds((b % 2) * KG, KG)]

    def didx(p, t):
        return didx2[p].at[t // 2, pl.ds((t % 2) * KG, KG)]

    idx_load(0)
    idx_wait(0)
    for b in range(NB):
        pltpu.async_copy(hs_hbm.at[gidx(0, 0, b)], rows_v.at[b], gsem[b])
    idx_load(1)
    pltpu.make_async_copy(zeros_hbm, acc.at[pl.ds(base, ROWS_T)], zsem).wait()
    plsc.subcore_barrier()

    def quarter_pass(q):
        p = q % 2

        def super_step(g, carry):
            for b in range(NB):
                t = g * NB + b
                pltpu.make_async_copy(
                    hs_hbm.at[gidx(p, g, b)], rows_v.at[b], gsem[b]).wait()
                pltpu.async_copy(rows_v.at[b], acc.at[didx(p, t)],
                                 ssem[b], add=True)
            for b in range(NB):
                t = g * NB + b
                pltpu.make_async_copy(
                    rows_v.at[b], acc.at[didx(p, t)], ssem[b]).wait()

                @pl.when(g < G_SUP - 1)
                def _():
                    pltpu.async_copy(
                        hs_hbm.at[gidx(p, g + 1, b)], rows_v.at[b], gsem[b])
            return carry

        lax.fori_loop(0, G_SUP - 1, super_step, 0)
        g = G_SUP - 1
        pn = 1 - p
        if q + 1 < NQ:
            idx_wait(q + 1)
        for b in range(NB):
            t = g * NB + b
            pltpu.make_async_copy(
                hs_hbm.at[gidx(p, g, b)], rows_v.at[b], gsem[b]).wait()
            pltpu.async_copy(rows_v.at[b], acc.at[didx(p, t)],
                             ssem[b], add=True)
        for b in range(NB):
            t = g * NB + b
            pltpu.make_async_copy(
                rows_v.at[b], acc.at[didx(p, t)], ssem[b]).wait()
            if q + 1 < NQ:
                pltpu.async_copy(
                    hs_hbm.at[gidx(pn, 0, b)], rows_v.at[b], gsem[b])
        if q + 2 < NQ:
            idx_load(q + 2)

    for q in range(NQ):
        quarter_pass(q)
    plsc.subcore_barrier()
    for i in range(ROWS_T // FK):
        pltpu.async_copy(acc.at[pl.ds(base + i * FK, FK)],
                         out_hbm.at[c, pl.ds(base + i * FK, FK)], fsem)
    for i in range(ROWS_T // FK):
        pltpu.make_async_copy(acc.at[pl.ds(base + i * FK, FK)],
                              out_hbm.at[c, pl.ds(base + i * FK, FK)], fsem).wait()


@functools.cache
def _sc_kernels():
    mesh = plsc.VectorSubcoreMesh(core_axis_name="c", subcore_axis_name="s")
    deg = pl.kernel(
        _deg_body,
        out_type=jax.ShapeDtypeStruct((NC, 2, N_PAD), jnp.float32),
        mesh=mesh,
        scratch_types=[
            pltpu.VMEM((C_CH, K), jnp.int32),
            pltpu.VMEM((C_CH, K), jnp.int32),
            pltpu.VMEM((K,), jnp.float32),
            pltpu.VMEM((ROWS_T,), jnp.float32),
            pltpu.VMEM_SHARED((N_PAD,), jnp.float32),
            pltpu.VMEM_SHARED((N_PAD,), jnp.float32),
            pltpu.SemaphoreType.DMA,
            pltpu.SemaphoreType.DMA,
        ],
    )
    agg = pl.kernel(
        _agg_body,
        out_type=jax.ShapeDtypeStruct((NC, N_PAD, D), jnp.float32),
        mesh=mesh,
        scratch_types=[
            pltpu.VMEM((C_Q, K), jnp.int32),
            pltpu.VMEM((C_Q, K), jnp.int32),
            pltpu.VMEM((C_Q, K), jnp.int32),
            pltpu.VMEM((C_Q, K), jnp.int32),
            pltpu.VMEM((NB, KG, D), jnp.float32),
            pltpu.VMEM_SHARED((N_PAD, D), jnp.float32),
        ] + [pltpu.SemaphoreType.DMA] * 14,
    )
    return deg, agg


def _mm1_body(deg_ref, x_ref, w_ref, o_ref):
    nsrc = lax.rsqrt(jnp.maximum(deg_ref[0, 0] + deg_ref[1, 0], 1.0))
    o_ref[...] = (
        jnp.dot(x_ref[...], w_ref[...], preferred_element_type=jnp.float32) * nsrc
    )


def _mid_body(deg_ref, p_ref, b1_ref, w2_ref, o_ref):
    nsrc = lax.rsqrt(jnp.maximum(deg_ref[0, 0] + deg_ref[1, 0], 1.0))
    ndst = lax.rsqrt(jnp.maximum(deg_ref[0, 1] + deg_ref[1, 1], 1.0))
    agg = p_ref[0] + p_ref[1]
    h1 = jnp.maximum(agg * ndst + b1_ref[...], 0.0)
    o_ref[...] = (
        jnp.dot(h1, w2_ref[...], preferred_element_type=jnp.float32) * nsrc
    )


def _tail_body(deg_ref, p_ref, b2_ref, wf1_ref, bf1_ref, wf2_ref, bf2_ref,
               o_ref, mx_ref):
    r = pl.program_id(0)
    ndst = lax.rsqrt(jnp.maximum(deg_ref[0, 1] + deg_ref[1, 1], 1.0))
    h2 = jnp.maximum((p_ref[0] + p_ref[1]) * ndst + b2_ref[...], 0.0)
    mx = jnp.max(h2, axis=0, keepdims=True)

    @pl.when(r == 0)
    def _():
        mx_ref[...] = mx

    @pl.when(r > 0)
    def _():
        mx_ref[...] = jnp.maximum(mx_ref[...], mx)

    @pl.when(r == GR - 1)
    def _():
        pooled = mx_ref[...]
        h = jnp.maximum(
            jnp.dot(pooled, wf1_ref[...], preferred_element_type=jnp.float32)
            + bf1_ref[...], 0.0)
        o_ref[...] = (
            jnp.dot(h, wf2_ref[...], preferred_element_type=jnp.float32)
            + bf2_ref[...])


_DEG_SPEC = pl.BlockSpec((NC, 2, BR, 1), lambda r: (0, 0, r, 0))
_W_SPEC = pl.BlockSpec((D, D), lambda r: (0, 0))
_ROW_SPEC = pl.BlockSpec((BR, D), lambda r: (r, 0))
_PARTS_SPEC = pl.BlockSpec((NC, BR, D), lambda r: (0, r, 0))
_B_SPEC = pl.BlockSpec((1, D), lambda r: (0, 0))

_mm1_call = pl.pallas_call(
    _mm1_body,
    grid=(GR,),
    in_specs=[_DEG_SPEC, _ROW_SPEC, _W_SPEC],
    out_specs=_ROW_SPEC,
    out_shape=jax.ShapeDtypeStruct((N, D), jnp.float32),
)

_mid_call = pl.pallas_call(
    _mid_body,
    grid=(GR,),
    in_specs=[_DEG_SPEC, _PARTS_SPEC, _B_SPEC, _W_SPEC],
    out_specs=_ROW_SPEC,
    out_shape=jax.ShapeDtypeStruct((N, D), jnp.float32),
)

_tail_call = pl.pallas_call(
    _tail_body,
    grid=(GR,),
    in_specs=[
        _DEG_SPEC,
        _PARTS_SPEC,
        _B_SPEC,
        pl.BlockSpec((D, HID2), lambda r: (0, 0)),
        pl.BlockSpec((1, HID2), lambda r: (0, 0)),
        pl.BlockSpec((HID2, EMB), lambda r: (0, 0)),
        pl.BlockSpec((1, EMB), lambda r: (0, 0)),
    ],
    out_specs=pl.BlockSpec((1, EMB), lambda r: (0, 0)),
    out_shape=jax.ShapeDtypeStruct((1, EMB), jnp.float32),
    scratch_shapes=[pltpu.VMEM((1, D), jnp.float32)],
)


def kernel(x, edge_index, W1, b1, W2, b2, Wfc1, bfc1, Wfc2, bfc2):
    src = edge_index[0].astype(jnp.int32)
    dst = edge_index[1].astype(jnp.int32)
    n_extra = E_PAD - src.shape[0]
    spread = np.arange(n_extra, dtype=np.int32) % 240
    pad_hi = jnp.asarray(N + spread)
    pad_lo = jnp.asarray(spread)
    src_deg = jnp.concatenate([src, pad_hi]).reshape(NW, C_CH, K)
    dst_p = jnp.concatenate([dst, pad_hi]).reshape(NW, C_CH, K)
    src_a = jnp.concatenate([src, pad_lo]).reshape(NW, C_CH, K)
    dst_a = dst_p

    deg_k, agg_k = _sc_kernels()
    zrows = jnp.zeros((ROWS_T, D), jnp.float32)
    deg4 = deg_k(src_deg, dst_p).reshape(NC, 2, N_PAD, 1)
    hs1 = _mm1_call(deg4, x, W1)
    parts1 = agg_k(hs1, src_a, dst_a, zrows)
    hs2 = _mid_call(deg4, parts1, b1.reshape(1, D), W2)
    parts2 = agg_k(hs2, src_a, dst_a, zrows)
    return _tail_call(deg4, parts2, b2.reshape(1, D), Wfc1,
                      bfc1.reshape(1, HID2), Wfc2, bfc2.reshape(1, EMB))
